# baseline (speedup 1.0000x reference)
import functools

import jax
import jax.numpy as jnp
from jax import lax
from jax.experimental import pallas as pl
from jax.experimental.pallas import tpu as pltpu
from jax.experimental.pallas import tpu_sc as plsc

DIM = 1024
HID = 2048
E = 8
T = 2048
BT = 256
TILE = 256
CAP = T
NTILES = CAP // TILE
NW = 32
PER_W = T // NW


def _router_kernel(x_ref, rw_ref, xbf_ref, s1_ref, s2_ref, w1_ref, w2_ref,
                   cnt_ref, carry_ref):
    b = pl.program_id(0)

    @pl.when(b == 0)
    def _():
        carry_ref[...] = jnp.zeros_like(carry_ref)

    xb = x_ref[...]
    xbf_ref[...] = xb.astype(jnp.bfloat16)

    logits = jnp.dot(xb, rw_ref[...].T, preferred_element_type=jnp.float32)
    eidx = lax.broadcasted_iota(jnp.int32, logits.shape, 1)
    m1 = jnp.max(logits, axis=1, keepdims=True)
    i1 = jnp.min(jnp.where(logits == m1, eidx, E), axis=1, keepdims=True)
    masked = jnp.where(eidx == i1, -jnp.inf, logits)
    m2 = jnp.max(masked, axis=1, keepdims=True)
    i2 = jnp.min(jnp.where(masked == m2, eidx, E), axis=1, keepdims=True)
    denom = jnp.sum(jnp.exp(logits - m1), axis=1, keepdims=True)
    p1 = 1.0 / denom
    p2 = jnp.exp(m2 - m1) / denom
    s = p1 + p2 + 1e-8
    w1_ref[0] = jnp.broadcast_to(p1 / s, (BT, 16))
    w2_ref[0] = jnp.broadcast_to(p2 / s, (BT, 16))

    sel1 = (eidx == i1).astype(jnp.float32)
    sel2 = (eidx == i2).astype(jnp.float32)
    sel = sel1 + sel2
    ri = lax.broadcasted_iota(jnp.int32, (BT, BT), 0)
    ci = lax.broadcasted_iota(jnp.int32, (BT, BT), 1)
    ltri = (ri > ci).astype(jnp.bfloat16)
    pos = jnp.dot(ltri, sel.astype(jnp.bfloat16),
                  preferred_element_type=jnp.float32)
    pos = pos + carry_ref[...]
    base = (eidx * CAP).astype(jnp.float32)
    slotf = base + pos
    s1_ref[0] = jnp.sum(sel1 * slotf, axis=1, keepdims=True).astype(jnp.int32)
    s2_ref[0] = jnp.sum(sel2 * slotf, axis=1, keepdims=True).astype(jnp.int32)
    carry_ref[...] += jnp.sum(sel, axis=0, keepdims=True)

    @pl.when(b == pl.num_programs(0) - 1)
    def _():
        cnt_ref[...] = carry_ref[...].astype(jnp.int32)


def _run_router(x_flat, router_w):
    nb = T // BT
    return pl.pallas_call(
        _router_kernel,
        grid=(nb,),
        in_specs=[
            pl.BlockSpec((BT, DIM), lambda b: (b, 0)),
            pl.BlockSpec((E, DIM), lambda b: (0, 0)),
        ],
        out_specs=[
            pl.BlockSpec((BT, DIM), lambda b: (b, 0)),
            pl.BlockSpec((1, BT, 1), lambda b: (b, 0, 0)),
            pl.BlockSpec((1, BT, 1), lambda b: (b, 0, 0)),
            pl.BlockSpec((1, BT, 16), lambda b: (b, 0, 0)),
            pl.BlockSpec((1, BT, 16), lambda b: (b, 0, 0)),
            pl.BlockSpec((1, E), lambda b: (0, 0)),
        ],
        out_shape=[
            jax.ShapeDtypeStruct((T, DIM), jnp.bfloat16),
            jax.ShapeDtypeStruct((nb, BT, 1), jnp.int32),
            jax.ShapeDtypeStruct((nb, BT, 1), jnp.int32),
            jax.ShapeDtypeStruct((nb, BT, 16), jnp.float32),
            jax.ShapeDtypeStruct((nb, BT, 16), jnp.float32),
            jax.ShapeDtypeStruct((1, E), jnp.int32),
        ],
        scratch_shapes=[pltpu.VMEM((1, E), jnp.float32)],
        compiler_params=pltpu.CompilerParams(
            dimension_semantics=("arbitrary",),
        ),
    )(x_flat, router_w)


def _run_dispatch(x_i32, s1, s2):
    mesh = plsc.VectorSubcoreMesh(core_axis_name="c", subcore_axis_name="s")

    @functools.partial(
        pl.kernel,
        mesh=mesh,
        out_type=jax.ShapeDtypeStruct((E * CAP, DIM // 2), jnp.int32),
        scratch_types=[
            pltpu.VMEM((PER_W,), jnp.int32),
            pltpu.VMEM((PER_W,), jnp.int32),
            pltpu.VMEM((PER_W, DIM // 2), jnp.int32),
            pltpu.SemaphoreType.DMA,
        ],
    )
    def dispatch(x_hbm, s1_hbm, s2_hbm, xs_hbm, i1_v, i2_v, xr_v, sem):
        wid = lax.axis_index("s") * 2 + lax.axis_index("c")
        base = wid * PER_W
        pltpu.sync_copy(s1_hbm.at[pl.ds(base, PER_W)], i1_v)
        pltpu.sync_copy(s2_hbm.at[pl.ds(base, PER_W)], i2_v)
        pltpu.sync_copy(x_hbm.at[pl.ds(base, PER_W)], xr_v)
        c1 = pltpu.async_copy(xr_v, xs_hbm.at[i1_v], sem)
        c2 = pltpu.async_copy(xr_v, xs_hbm.at[i2_v], sem)
        c1.wait()
        c2.wait()

    return dispatch(x_i32, s1, s2)


def _mlp_kernel(cnt_ref, xs_ref, fc_ref, pj_ref, ys_ref, fcb_ref, pjb_ref):
    e = pl.program_id(0)
    t = pl.program_id(1)
    c_e = cnt_ref[e]

    @pl.when(jnp.logical_and(t == 0, c_e > 0))
    def _():
        fcb_ref[...] = fc_ref[0].astype(jnp.bfloat16)
        pjb_ref[...] = pj_ref[0].astype(jnp.bfloat16)

    @pl.when(t * TILE < c_e)
    def _():
        h = jnp.dot(xs_ref[...], fcb_ref[...].T,
                    preferred_element_type=jnp.float32)
        h = jnp.square(jnp.maximum(h, 0.0))
        y = jnp.dot(h.astype(jnp.bfloat16), pjb_ref[...].T,
                    preferred_element_type=jnp.float32)
        ys_ref[...] = y


def _run_mlp(counts, xs, fc_w, proj_w):
    def tile_map(e, t, cnt):
        ntile = jnp.maximum(lax.div(cnt[e] + (TILE - 1), TILE), 1)
        return (e * NTILES + jnp.minimum(t, ntile - 1), 0)

    grid_spec = pltpu.PrefetchScalarGridSpec(
        num_scalar_prefetch=1,
        grid=(E, NTILES),
        in_specs=[
            pl.BlockSpec((TILE, DIM), tile_map),
            pl.BlockSpec((1, HID, DIM), lambda e, t, cnt: (e, 0, 0)),
            pl.BlockSpec((1, DIM, HID), lambda e, t, cnt: (e, 0, 0)),
        ],
        out_specs=pl.BlockSpec((TILE, DIM), tile_map),
        scratch_shapes=[
            pltpu.VMEM((HID, DIM), jnp.bfloat16),
            pltpu.VMEM((DIM, HID), jnp.bfloat16),
        ],
    )
    return pl.pallas_call(
        _mlp_kernel,
        grid_spec=grid_spec,
        out_shape=jax.ShapeDtypeStruct((E * CAP, DIM), jnp.float32),
        compiler_params=pltpu.CompilerParams(
            dimension_semantics=("arbitrary", "arbitrary"),
        ),
    )(counts, xs, fc_w, proj_w)


_SC_CHUNK = 32


def _run_combine(ys, s1, s2, w1b, w2b):
    mesh = plsc.VectorSubcoreMesh(core_axis_name="c", subcore_axis_name="s")

    @functools.partial(
        pl.kernel,
        mesh=mesh,
        out_type=jax.ShapeDtypeStruct((T, DIM), jnp.float32),
        scratch_types=[
            pltpu.VMEM((_SC_CHUNK,), jnp.int32),
            pltpu.VMEM((_SC_CHUNK,), jnp.int32),
            pltpu.VMEM((_SC_CHUNK, DIM), jnp.float32),
            pltpu.VMEM((_SC_CHUNK, DIM), jnp.float32),
            pltpu.VMEM((_SC_CHUNK, 16), jnp.float32),
            pltpu.VMEM((_SC_CHUNK, 16), jnp.float32),
            pltpu.SemaphoreType.DMA,
        ],
    )
    def combine(ys_hbm, s1_hbm, s2_hbm, w1_hbm, w2_hbm, out_hbm,
                i1_v, i2_v, ra_v, rb_v, wa_v, wb_v, sem):
        wid = lax.axis_index("s") * 2 + lax.axis_index("c")
        base = wid * PER_W
        for chunk in range(PER_W // _SC_CHUNK):
            off = base + chunk * _SC_CHUNK
            pltpu.sync_copy(s1_hbm.at[pl.ds(off, _SC_CHUNK)], i1_v)
            pltpu.sync_copy(s2_hbm.at[pl.ds(off, _SC_CHUNK)], i2_v)
            pltpu.sync_copy(w1_hbm.at[pl.ds(off, _SC_CHUNK)], wa_v)
            pltpu.sync_copy(w2_hbm.at[pl.ds(off, _SC_CHUNK)], wb_v)
            ca = pltpu.async_copy(ys_hbm.at[i1_v], ra_v, sem)
            cb = pltpu.async_copy(ys_hbm.at[i2_v], rb_v, sem)
            ca.wait()
            cb.wait()

            @pl.loop(0, _SC_CHUNK)
            def _(r):
                wa = wa_v.at[r][...]
                wb = wb_v.at[r][...]

                @pl.loop(0, DIM // 16)
                def _(c):
                    sl = (r, pl.ds(c * 16, 16))
                    ra_v.at[*sl][...] = (ra_v.at[*sl][...] * wa
                                         + rb_v.at[*sl][...] * wb)

            pltpu.sync_copy(ra_v, out_hbm.at[pl.ds(off, _SC_CHUNK)])

    return combine(ys, s1, s2, w1b, w2b)


def kernel(x, router_w, fc_w, proj_w):
    bsz, seq_len, dim = x.shape
    x_flat = x.reshape(-1, dim)
    x_bf, s1, s2, w1b, w2b, counts = _run_router(x_flat, router_w)
    s1f = s1.reshape(T)
    s2f = s2.reshape(T)
    x_i32 = lax.bitcast_convert_type(x_bf.reshape(T, DIM // 2, 2), jnp.int32)
    xs_i32 = _run_dispatch(x_i32, s1f, s2f)
    xs = lax.bitcast_convert_type(xs_i32, jnp.bfloat16).reshape(E * CAP, DIM)
    ys = _run_mlp(counts.reshape(E), xs, fc_w, proj_w)
    out = _run_combine(ys, s1f, s2f, w1b.reshape(T, 16), w2b.reshape(T, 16))
    return out.reshape(bsz, seq_len, dim), jnp.float32(0.0)


# speedup vs baseline: 2.4248x; 2.4248x over previous
import functools

import jax
import jax.numpy as jnp
from jax import lax
from jax.experimental import pallas as pl
from jax.experimental.pallas import tpu as pltpu
from jax.experimental.pallas import tpu_sc as plsc

DIM = 1024
HID = 2048
E = 8
T = 2048
BT = 256
TILE = 256
CAP = T
NTILES = CAP // TILE
NW = 32
PER_W = T // NW


def _router_kernel(x_ref, rw_ref, s1_ref, s2_ref, w1_ref, w2_ref,
                   cnt_ref, carry_ref):
    b = pl.program_id(0)

    @pl.when(b == 0)
    def _():
        carry_ref[...] = jnp.zeros_like(carry_ref)

    xb = x_ref[...]
    logits = jnp.dot(xb, rw_ref[...].T, preferred_element_type=jnp.float32)
    eidx = lax.broadcasted_iota(jnp.int32, logits.shape, 1)
    m1 = jnp.max(logits, axis=1, keepdims=True)
    i1 = jnp.min(jnp.where(logits == m1, eidx, E), axis=1, keepdims=True)
    masked = jnp.where(eidx == i1, -jnp.inf, logits)
    m2 = jnp.max(masked, axis=1, keepdims=True)
    i2 = jnp.min(jnp.where(masked == m2, eidx, E), axis=1, keepdims=True)
    denom = jnp.sum(jnp.exp(logits - m1), axis=1, keepdims=True)
    p1 = 1.0 / denom
    p2 = jnp.exp(m2 - m1) / denom
    s = p1 + p2 + 1e-8
    w1_ref[0] = jnp.broadcast_to(p1 / s, (BT, 16))
    w2_ref[0] = jnp.broadcast_to(p2 / s, (BT, 16))

    sel1 = (eidx == i1).astype(jnp.float32)
    sel2 = (eidx == i2).astype(jnp.float32)
    sel = sel1 + sel2
    ri = lax.broadcasted_iota(jnp.int32, (BT, BT), 0)
    ci = lax.broadcasted_iota(jnp.int32, (BT, BT), 1)
    ltri = (ri > ci).astype(jnp.bfloat16)
    pos = jnp.dot(ltri, sel.astype(jnp.bfloat16),
                  preferred_element_type=jnp.float32)
    pos = pos + carry_ref[...]
    base = (eidx * CAP).astype(jnp.float32)
    slotf = base + pos
    s1_ref[0] = jnp.sum(sel1 * slotf, axis=1, keepdims=True).astype(jnp.int32)
    s2_ref[0] = jnp.sum(sel2 * slotf, axis=1, keepdims=True).astype(jnp.int32)
    carry_ref[...] += jnp.sum(sel, axis=0, keepdims=True)

    @pl.when(b == pl.num_programs(0) - 1)
    def _():
        cnt_ref[...] = carry_ref[...].astype(jnp.int32)


def _run_router(x_flat, router_w):
    nb = T // BT
    return pl.pallas_call(
        _router_kernel,
        grid=(nb,),
        in_specs=[
            pl.BlockSpec((BT, DIM), lambda b: (b, 0)),
            pl.BlockSpec((E, DIM), lambda b: (0, 0)),
        ],
        out_specs=[
            pl.BlockSpec((1, BT, 1), lambda b: (b, 0, 0)),
            pl.BlockSpec((1, BT, 1), lambda b: (b, 0, 0)),
            pl.BlockSpec((1, BT, 16), lambda b: (b, 0, 0)),
            pl.BlockSpec((1, BT, 16), lambda b: (b, 0, 0)),
            pl.BlockSpec((1, E), lambda b: (0, 0)),
        ],
        out_shape=[
            jax.ShapeDtypeStruct((nb, BT, 1), jnp.int32),
            jax.ShapeDtypeStruct((nb, BT, 1), jnp.int32),
            jax.ShapeDtypeStruct((nb, BT, 16), jnp.float32),
            jax.ShapeDtypeStruct((nb, BT, 16), jnp.float32),
            jax.ShapeDtypeStruct((1, E), jnp.int32),
        ],
        scratch_shapes=[pltpu.VMEM((1, E), jnp.float32)],
        compiler_params=pltpu.CompilerParams(
            dimension_semantics=("arbitrary",),
        ),
    )(x_flat, router_w)


def _run_dispatch(x_flat, s1, s2):
    mesh = plsc.VectorSubcoreMesh(core_axis_name="c", subcore_axis_name="s")

    @functools.partial(
        pl.kernel,
        mesh=mesh,
        out_type=jax.ShapeDtypeStruct((E * CAP, DIM), jnp.float32),
        scratch_types=[
            pltpu.VMEM((PER_W,), jnp.int32),
            pltpu.VMEM((PER_W,), jnp.int32),
            pltpu.VMEM((PER_W, DIM), jnp.float32),
            pltpu.SemaphoreType.DMA,
        ],
    )
    def dispatch(x_hbm, s1_hbm, s2_hbm, xs_hbm, i1_v, i2_v, xr_v, sem):
        wid = lax.axis_index("s") * 2 + lax.axis_index("c")
        base = wid * PER_W
        pltpu.sync_copy(s1_hbm.at[pl.ds(base, PER_W)], i1_v)
        pltpu.sync_copy(s2_hbm.at[pl.ds(base, PER_W)], i2_v)
        pltpu.sync_copy(x_hbm.at[pl.ds(base, PER_W)], xr_v)
        c1 = pltpu.async_copy(xr_v, xs_hbm.at[i1_v], sem)
        c2 = pltpu.async_copy(xr_v, xs_hbm.at[i2_v], sem)
        c1.wait()
        c2.wait()

    return dispatch(x_flat, s1, s2)


def _mlp_kernel(cnt_ref, xs_ref, fc_ref, pj_ref, ys_ref, fcb_ref, pjb_ref):
    e = pl.program_id(0)
    t = pl.program_id(1)
    c_e = cnt_ref[e]

    @pl.when(jnp.logical_and(t == 0, c_e > 0))
    def _():
        fcb_ref[...] = fc_ref[0].astype(jnp.bfloat16)
        pjb_ref[...] = pj_ref[0].astype(jnp.bfloat16)

    @pl.when(t * TILE < c_e)
    def _():
        h = jnp.dot(xs_ref[...].astype(jnp.bfloat16), fcb_ref[...].T,
                    preferred_element_type=jnp.float32)
        h = jnp.square(jnp.maximum(h, 0.0))
        y = jnp.dot(h.astype(jnp.bfloat16), pjb_ref[...].T,
                    preferred_element_type=jnp.float32)
        ys_ref[...] = y


def _run_mlp(counts, xs, fc_w, proj_w):
    def tile_map(e, t, cnt):
        ntile = jnp.maximum(lax.div(cnt[e] + (TILE - 1), TILE), 1)
        return (e * NTILES + jnp.minimum(t, ntile - 1), 0)

    grid_spec = pltpu.PrefetchScalarGridSpec(
        num_scalar_prefetch=1,
        grid=(E, NTILES),
        in_specs=[
            pl.BlockSpec((TILE, DIM), tile_map),
            pl.BlockSpec((1, HID, DIM), lambda e, t, cnt: (e, 0, 0)),
            pl.BlockSpec((1, DIM, HID), lambda e, t, cnt: (e, 0, 0)),
        ],
        out_specs=pl.BlockSpec((TILE, DIM), tile_map),
        scratch_shapes=[
            pltpu.VMEM((HID, DIM), jnp.bfloat16),
            pltpu.VMEM((DIM, HID), jnp.bfloat16),
        ],
    )
    return pl.pallas_call(
        _mlp_kernel,
        grid_spec=grid_spec,
        out_shape=jax.ShapeDtypeStruct((E * CAP, DIM), jnp.float32),
        compiler_params=pltpu.CompilerParams(
            dimension_semantics=("arbitrary", "arbitrary"),
        ),
    )(counts, xs, fc_w, proj_w)


_SC_CHUNK = 32


def _run_combine(ys, s1, s2, w1b, w2b):
    mesh = plsc.VectorSubcoreMesh(core_axis_name="c", subcore_axis_name="s")

    @functools.partial(
        pl.kernel,
        mesh=mesh,
        out_type=jax.ShapeDtypeStruct((T, DIM), jnp.float32),
        scratch_types=[
            pltpu.VMEM((_SC_CHUNK,), jnp.int32),
            pltpu.VMEM((_SC_CHUNK,), jnp.int32),
            pltpu.VMEM((_SC_CHUNK, DIM), jnp.float32),
            pltpu.VMEM((_SC_CHUNK, DIM), jnp.float32),
            pltpu.VMEM((_SC_CHUNK, 16), jnp.float32),
            pltpu.VMEM((_SC_CHUNK, 16), jnp.float32),
            pltpu.SemaphoreType.DMA,
        ],
    )
    def combine(ys_hbm, s1_hbm, s2_hbm, w1_hbm, w2_hbm, out_hbm,
                i1_v, i2_v, ra_v, rb_v, wa_v, wb_v, sem):
        wid = lax.axis_index("s") * 2 + lax.axis_index("c")
        base = wid * PER_W
        for chunk in range(PER_W // _SC_CHUNK):
            off = base + chunk * _SC_CHUNK
            pltpu.sync_copy(s1_hbm.at[pl.ds(off, _SC_CHUNK)], i1_v)
            pltpu.sync_copy(s2_hbm.at[pl.ds(off, _SC_CHUNK)], i2_v)
            pltpu.sync_copy(w1_hbm.at[pl.ds(off, _SC_CHUNK)], wa_v)
            pltpu.sync_copy(w2_hbm.at[pl.ds(off, _SC_CHUNK)], wb_v)
            ca = pltpu.async_copy(ys_hbm.at[i1_v], ra_v, sem)
            cb = pltpu.async_copy(ys_hbm.at[i2_v], rb_v, sem)
            ca.wait()
            cb.wait()

            @pl.loop(0, _SC_CHUNK)
            def _(r):
                wa = wa_v.at[r][...]
                wb = wb_v.at[r][...]

                @pl.loop(0, DIM // 16)
                def _(c):
                    sl = (r, pl.ds(c * 16, 16))
                    ra_v.at[*sl][...] = (ra_v.at[*sl][...] * wa
                                         + rb_v.at[*sl][...] * wb)

            pltpu.sync_copy(ra_v, out_hbm.at[pl.ds(off, _SC_CHUNK)])

    return combine(ys, s1, s2, w1b, w2b)


def kernel(x, router_w, fc_w, proj_w):
    bsz, seq_len, dim = x.shape
    x_flat = x.reshape(-1, dim)
    s1, s2, w1b, w2b, counts = _run_router(x_flat, router_w)
    s1f = s1.reshape(T)
    s2f = s2.reshape(T)
    xs = _run_dispatch(x_flat, s1f, s2f)
    ys = _run_mlp(counts.reshape(E), xs, fc_w, proj_w)
    out = _run_combine(ys, s1f, s2f, w1b.reshape(T, 16), w2b.reshape(T, 16))
    return out.reshape(bsz, seq_len, dim), jnp.float32(0.0)


# speedup vs baseline: 2.4935x; 1.0284x over previous
import functools

import jax
import jax.numpy as jnp
from jax import lax
from jax.experimental import pallas as pl
from jax.experimental.pallas import tpu as pltpu
from jax.experimental.pallas import tpu_sc as plsc

DIM = 1024
HID = 2048
E = 8
T = 2048
BT = 256
TILE = 256
CAP = T
NTILES = CAP // TILE
NW = 32
PER_W = T // NW


def _router_kernel(x_ref, rw_ref, s1_ref, s2_ref, w1_ref, w2_ref,
                   cnt_ref, carry_ref):
    b = pl.program_id(0)

    @pl.when(b == 0)
    def _():
        carry_ref[...] = jnp.zeros_like(carry_ref)

    xb = x_ref[...]
    logits = jnp.dot(xb, rw_ref[...].T, preferred_element_type=jnp.float32)
    eidx = lax.broadcasted_iota(jnp.int32, logits.shape, 1)
    m1 = jnp.max(logits, axis=1, keepdims=True)
    i1 = jnp.min(jnp.where(logits == m1, eidx, E), axis=1, keepdims=True)
    masked = jnp.where(eidx == i1, -jnp.inf, logits)
    m2 = jnp.max(masked, axis=1, keepdims=True)
    i2 = jnp.min(jnp.where(masked == m2, eidx, E), axis=1, keepdims=True)
    denom = jnp.sum(jnp.exp(logits - m1), axis=1, keepdims=True)
    p1 = 1.0 / denom
    p2 = jnp.exp(m2 - m1) / denom
    s = p1 + p2 + 1e-8
    w1_ref[0] = jnp.broadcast_to(p1 / s, (BT, 16))
    w2_ref[0] = jnp.broadcast_to(p2 / s, (BT, 16))

    sel1 = (eidx == i1).astype(jnp.float32)
    sel2 = (eidx == i2).astype(jnp.float32)
    sel = sel1 + sel2
    ri = lax.broadcasted_iota(jnp.int32, (BT, BT), 0)
    ci = lax.broadcasted_iota(jnp.int32, (BT, BT), 1)
    ltri = (ri > ci).astype(jnp.bfloat16)
    pos = jnp.dot(ltri, sel.astype(jnp.bfloat16),
                  preferred_element_type=jnp.float32)
    pos = pos + carry_ref[...]
    base = (eidx * CAP).astype(jnp.float32)
    slotf = base + pos
    s1_ref[0] = jnp.sum(sel1 * slotf, axis=1, keepdims=True).astype(jnp.int32)
    s2_ref[0] = jnp.sum(sel2 * slotf, axis=1, keepdims=True).astype(jnp.int32)
    carry_ref[...] += jnp.sum(sel, axis=0, keepdims=True)

    @pl.when(b == pl.num_programs(0) - 1)
    def _():
        cnt_ref[...] = carry_ref[...].astype(jnp.int32)


def _run_router(x_flat, router_w):
    nb = T // BT
    return pl.pallas_call(
        _router_kernel,
        grid=(nb,),
        in_specs=[
            pl.BlockSpec((BT, DIM), lambda b: (b, 0)),
            pl.BlockSpec((E, DIM), lambda b: (0, 0)),
        ],
        out_specs=[
            pl.BlockSpec((1, BT, 1), lambda b: (b, 0, 0)),
            pl.BlockSpec((1, BT, 1), lambda b: (b, 0, 0)),
            pl.BlockSpec((1, BT, 16), lambda b: (b, 0, 0)),
            pl.BlockSpec((1, BT, 16), lambda b: (b, 0, 0)),
            pl.BlockSpec((1, E), lambda b: (0, 0)),
        ],
        out_shape=[
            jax.ShapeDtypeStruct((nb, BT, 1), jnp.int32),
            jax.ShapeDtypeStruct((nb, BT, 1), jnp.int32),
            jax.ShapeDtypeStruct((nb, BT, 16), jnp.float32),
            jax.ShapeDtypeStruct((nb, BT, 16), jnp.float32),
            jax.ShapeDtypeStruct((1, E), jnp.int32),
        ],
        scratch_shapes=[pltpu.VMEM((1, E), jnp.float32)],
        compiler_params=pltpu.CompilerParams(
            dimension_semantics=("arbitrary",),
        ),
    )(x_flat, router_w)


def _run_dispatch(x_flat, s1, s2):
    mesh = plsc.VectorSubcoreMesh(core_axis_name="c", subcore_axis_name="s")

    @functools.partial(
        pl.kernel,
        mesh=mesh,
        out_type=jax.ShapeDtypeStruct((E * CAP, DIM), jnp.float32),
        scratch_types=[
            pltpu.VMEM((PER_W,), jnp.int32),
            pltpu.VMEM((PER_W,), jnp.int32),
            pltpu.VMEM((PER_W, DIM), jnp.float32),
            pltpu.SemaphoreType.DMA,
        ],
    )
    def dispatch(x_hbm, s1_hbm, s2_hbm, xs_hbm, i1_v, i2_v, xr_v, sem):
        wid = lax.axis_index("s") * 2 + lax.axis_index("c")
        base = wid * PER_W
        pltpu.sync_copy(s1_hbm.at[pl.ds(base, PER_W)], i1_v)
        pltpu.sync_copy(s2_hbm.at[pl.ds(base, PER_W)], i2_v)
        pltpu.sync_copy(x_hbm.at[pl.ds(base, PER_W)], xr_v)
        c1 = pltpu.async_copy(xr_v, xs_hbm.at[i1_v], sem)
        c2 = pltpu.async_copy(xr_v, xs_hbm.at[i2_v], sem)
        c1.wait()
        c2.wait()

    return dispatch(x_flat, s1, s2)


def _mlp_kernel(cnt_ref, xs_ref, fc_ref, pj_ref, ys_ref, fcb_ref, pjb_ref):
    e = pl.program_id(0)
    t = pl.program_id(1)
    c_e = cnt_ref[e]

    @pl.when(jnp.logical_and(t == 0, c_e > 0))
    def _():
        fcb_ref[...] = fc_ref[0].astype(jnp.bfloat16)
        pjb_ref[...] = pj_ref[0].astype(jnp.bfloat16)

    @pl.when(t * TILE < c_e)
    def _():
        h = jnp.dot(xs_ref[...].astype(jnp.bfloat16), fcb_ref[...].T,
                    preferred_element_type=jnp.float32)
        h = jnp.square(jnp.maximum(h, 0.0))
        y = jnp.dot(h.astype(jnp.bfloat16), pjb_ref[...].T,
                    preferred_element_type=jnp.float32)
        ys_ref[...] = y


def _run_mlp(counts, xs, fc_w, proj_w):
    def tile_map(e, t, cnt):
        ntile = jnp.maximum(lax.div(cnt[e] + (TILE - 1), TILE), 1)
        return (e * NTILES + jnp.minimum(t, ntile - 1), 0)

    grid_spec = pltpu.PrefetchScalarGridSpec(
        num_scalar_prefetch=1,
        grid=(E, NTILES),
        in_specs=[
            pl.BlockSpec((TILE, DIM), tile_map),
            pl.BlockSpec((1, HID, DIM), lambda e, t, cnt: (e, 0, 0)),
            pl.BlockSpec((1, DIM, HID), lambda e, t, cnt: (e, 0, 0)),
        ],
        out_specs=pl.BlockSpec((TILE, DIM), tile_map),
        scratch_shapes=[
            pltpu.VMEM((HID, DIM), jnp.bfloat16),
            pltpu.VMEM((DIM, HID), jnp.bfloat16),
        ],
    )
    return pl.pallas_call(
        _mlp_kernel,
        grid_spec=grid_spec,
        out_shape=jax.ShapeDtypeStruct((E * CAP, DIM), jnp.float32),
        compiler_params=pltpu.CompilerParams(
            dimension_semantics=("arbitrary", "arbitrary"),
        ),
    )(counts, xs, fc_w, proj_w)


_SC_CHUNK = 16
_NCHUNK = PER_W // _SC_CHUNK


def _run_combine(ys, s1, s2, w1b, w2b):
    mesh = plsc.VectorSubcoreMesh(core_axis_name="c", subcore_axis_name="s")

    @functools.partial(
        pl.kernel,
        mesh=mesh,
        out_type=jax.ShapeDtypeStruct((T, DIM), jnp.float32),
        scratch_types=[
            pltpu.VMEM((PER_W,), jnp.int32),
            pltpu.VMEM((PER_W,), jnp.int32),
            pltpu.VMEM((PER_W, 16), jnp.float32),
            pltpu.VMEM((PER_W, 16), jnp.float32),
            pltpu.VMEM((2, _SC_CHUNK, DIM), jnp.float32),
            pltpu.VMEM((2, _SC_CHUNK, DIM), jnp.float32),
            pltpu.SemaphoreType.DMA,
            pltpu.SemaphoreType.DMA,
        ],
    )
    def combine(ys_hbm, s1_hbm, s2_hbm, w1_hbm, w2_hbm, out_hbm,
                i1_v, i2_v, wa_v, wb_v, ra_v, rb_v, sem0, sem1):
        wid = lax.axis_index("s") * 2 + lax.axis_index("c")
        base = wid * PER_W
        pltpu.sync_copy(s1_hbm.at[pl.ds(base, PER_W)], i1_v)
        pltpu.sync_copy(s2_hbm.at[pl.ds(base, PER_W)], i2_v)
        pltpu.sync_copy(w1_hbm.at[pl.ds(base, PER_W)], wa_v)
        pltpu.sync_copy(w2_hbm.at[pl.ds(base, PER_W)], wb_v)
        sems = (sem0, sem1)

        def issue(c):
            slot = c % 2
            ca = pltpu.async_copy(
                ys_hbm.at[i1_v.at[pl.ds(c * _SC_CHUNK, _SC_CHUNK)]],
                ra_v.at[slot], sems[slot])
            cb = pltpu.async_copy(
                ys_hbm.at[i2_v.at[pl.ds(c * _SC_CHUNK, _SC_CHUNK)]],
                rb_v.at[slot], sems[slot])
            return ca, cb

        pend = {0: issue(0), 1: issue(1)}
        for c in range(_NCHUNK):
            slot = c % 2
            ca, cb = pend.pop(c)
            ca.wait()
            cb.wait()

            @pl.loop(0, _SC_CHUNK)
            def _(r):
                wa = wa_v.at[c * _SC_CHUNK + r][...]
                wb = wb_v.at[c * _SC_CHUNK + r][...]

                @pl.loop(0, DIM // 16)
                def _(i):
                    sl = (slot, r, pl.ds(i * 16, 16))
                    ra_v.at[*sl][...] = (ra_v.at[*sl][...] * wa
                                         + rb_v.at[*sl][...] * wb)

            pltpu.sync_copy(ra_v.at[slot],
                            out_hbm.at[pl.ds(base + c * _SC_CHUNK, _SC_CHUNK)])
            if c + 2 < _NCHUNK:
                pend[c + 2] = issue(c + 2)

    return combine(ys, s1, s2, w1b, w2b)


def kernel(x, router_w, fc_w, proj_w):
    bsz, seq_len, dim = x.shape
    x_flat = x.reshape(-1, dim)
    s1, s2, w1b, w2b, counts = _run_router(x_flat, router_w)
    s1f = s1.reshape(T)
    s2f = s2.reshape(T)
    xs = _run_dispatch(x_flat, s1f, s2f)
    ys = _run_mlp(counts.reshape(E), xs, fc_w, proj_w)
    out = _run_combine(ys, s1f, s2f, w1b.reshape(T, 16), w2b.reshape(T, 16))
    return out.reshape(bsz, seq_len, dim), jnp.float32(0.0)


# speedup vs baseline: 2.7782x; 1.1142x over previous
import functools

import jax
import jax.numpy as jnp
from jax import lax
from jax.experimental import pallas as pl
from jax.experimental.pallas import tpu as pltpu
from jax.experimental.pallas import tpu_sc as plsc

DIM = 1024
HID = 2048
E = 8
T = 2048
BT = 256
TILE = 256
CAP = T
NTILES = CAP // TILE
GMAX = 23
NW = 32
PER_W = T // NW


def _router_kernel(x_ref, rw_ref, s1_ref, s2_ref, w1_ref, w2_ref,
                   te_ref, tt_ref, nt_ref, carry_ref):
    b = pl.program_id(0)

    @pl.when(b == 0)
    def _():
        carry_ref[...] = jnp.zeros_like(carry_ref)

    xb = x_ref[...]
    logits = jnp.dot(xb, rw_ref[...].T, preferred_element_type=jnp.float32)
    eidx = lax.broadcasted_iota(jnp.int32, logits.shape, 1)
    m1 = jnp.max(logits, axis=1, keepdims=True)
    i1 = jnp.min(jnp.where(logits == m1, eidx, E), axis=1, keepdims=True)
    masked = jnp.where(eidx == i1, -jnp.inf, logits)
    m2 = jnp.max(masked, axis=1, keepdims=True)
    i2 = jnp.min(jnp.where(masked == m2, eidx, E), axis=1, keepdims=True)
    denom = jnp.sum(jnp.exp(logits - m1), axis=1, keepdims=True)
    p1 = 1.0 / denom
    p2 = jnp.exp(m2 - m1) / denom
    s = p1 + p2 + 1e-8
    w1_ref[0] = jnp.broadcast_to(p1 / s, (BT, 16))
    w2_ref[0] = jnp.broadcast_to(p2 / s, (BT, 16))

    sel1 = (eidx == i1).astype(jnp.float32)
    sel2 = (eidx == i2).astype(jnp.float32)
    sel = sel1 + sel2
    ri = lax.broadcasted_iota(jnp.int32, (BT, BT), 0)
    ci = lax.broadcasted_iota(jnp.int32, (BT, BT), 1)
    ltri = (ri > ci).astype(jnp.bfloat16)
    pos = jnp.dot(ltri, sel.astype(jnp.bfloat16),
                  preferred_element_type=jnp.float32)
    pos = pos + carry_ref[...]
    base = (eidx * CAP).astype(jnp.float32)
    slotf = base + pos
    s1_ref[0] = jnp.sum(sel1 * slotf, axis=1, keepdims=True).astype(jnp.int32)
    s2_ref[0] = jnp.sum(sel2 * slotf, axis=1, keepdims=True).astype(jnp.int32)
    carry_ref[...] += jnp.sum(sel, axis=0, keepdims=True)

    @pl.when(b == pl.num_programs(0) - 1)
    def _():
        counts = carry_ref[...]
        nt = jnp.floor((counts + (TILE - 1)) * (1.0 / TILE))
        ut = (lax.broadcasted_iota(jnp.int32, (E, E), 0)
              <= lax.broadcasted_iota(jnp.int32, (E, E), 1))
        cume = jnp.dot(nt.astype(jnp.bfloat16), ut.astype(jnp.bfloat16),
                       preferred_element_type=jnp.float32)
        cums = cume - nt
        g_row = lax.broadcasted_iota(jnp.int32, (1, GMAX), 1).astype(jnp.float32)
        te = jnp.zeros((1, GMAX), jnp.float32)
        for e in range(E):
            te = te + (g_row >= cume[0, e]).astype(jnp.float32)
        te = jnp.minimum(te, float(E - 1))
        tt = g_row
        for e in range(E):
            tt = tt - jnp.where(te == e, cums[0, e], 0.0)
        tt = jnp.clip(tt, 0.0, float(NTILES - 1))
        ntot = cume[0, E - 1]
        last = jnp.maximum(ntot - 1.0, 0.0)
        te_last = jnp.sum(jnp.where(g_row == last, te, 0.0), axis=1,
                          keepdims=True)
        tt_last = jnp.sum(jnp.where(g_row == last, tt, 0.0), axis=1,
                          keepdims=True)
        active = g_row < ntot
        te_ref[...] = jnp.where(active, te, te_last).astype(jnp.int32)
        tt_ref[...] = jnp.where(active, tt, tt_last).astype(jnp.int32)
        nt_ref[...] = jnp.full((1, 1), ntot, jnp.float32).astype(jnp.int32)


def _run_router(x_flat, router_w):
    nb = T // BT
    return pl.pallas_call(
        _router_kernel,
        grid=(nb,),
        in_specs=[
            pl.BlockSpec((BT, DIM), lambda b: (b, 0)),
            pl.BlockSpec((E, DIM), lambda b: (0, 0)),
        ],
        out_specs=[
            pl.BlockSpec((1, BT, 1), lambda b: (b, 0, 0)),
            pl.BlockSpec((1, BT, 1), lambda b: (b, 0, 0)),
            pl.BlockSpec((1, BT, 16), lambda b: (b, 0, 0)),
            pl.BlockSpec((1, BT, 16), lambda b: (b, 0, 0)),
            pl.BlockSpec((1, GMAX), lambda b: (0, 0)),
            pl.BlockSpec((1, GMAX), lambda b: (0, 0)),
            pl.BlockSpec((1, 1), lambda b: (0, 0)),
        ],
        out_shape=[
            jax.ShapeDtypeStruct((nb, BT, 1), jnp.int32),
            jax.ShapeDtypeStruct((nb, BT, 1), jnp.int32),
            jax.ShapeDtypeStruct((nb, BT, 16), jnp.float32),
            jax.ShapeDtypeStruct((nb, BT, 16), jnp.float32),
            jax.ShapeDtypeStruct((1, GMAX), jnp.int32),
            jax.ShapeDtypeStruct((1, GMAX), jnp.int32),
            jax.ShapeDtypeStruct((1, 1), jnp.int32),
        ],
        scratch_shapes=[pltpu.VMEM((1, E), jnp.float32)],
        compiler_params=pltpu.CompilerParams(
            dimension_semantics=("arbitrary",),
        ),
    )(x_flat, router_w)


def _run_dispatch(x_flat, s1, s2):
    mesh = plsc.VectorSubcoreMesh(core_axis_name="c", subcore_axis_name="s")

    @functools.partial(
        pl.kernel,
        mesh=mesh,
        out_type=jax.ShapeDtypeStruct((E * CAP, DIM), jnp.float32),
        scratch_types=[
            pltpu.VMEM((PER_W,), jnp.int32),
            pltpu.VMEM((PER_W,), jnp.int32),
            pltpu.VMEM((PER_W, DIM), jnp.float32),
            pltpu.SemaphoreType.DMA,
        ],
    )
    def dispatch(x_hbm, s1_hbm, s2_hbm, xs_hbm, i1_v, i2_v, xr_v, sem):
        wid = lax.axis_index("s") * 2 + lax.axis_index("c")
        base = wid * PER_W
        pltpu.sync_copy(s1_hbm.at[pl.ds(base, PER_W)], i1_v)
        pltpu.sync_copy(s2_hbm.at[pl.ds(base, PER_W)], i2_v)
        pltpu.sync_copy(x_hbm.at[pl.ds(base, PER_W)], xr_v)
        c1 = pltpu.async_copy(xr_v, xs_hbm.at[i1_v], sem)
        c2 = pltpu.async_copy(xr_v, xs_hbm.at[i2_v], sem)
        c1.wait()
        c2.wait()

    return dispatch(x_flat, s1, s2)


def _mlp_kernel(te_ref, tt_ref, nt_ref, xs_ref, fc_ref, pj_ref, ys_ref,
                fcb_ref, pjb_ref):
    g = pl.program_id(0)

    @pl.when(g < nt_ref[0])
    def _():
        @pl.when(tt_ref[g] == 0)
        def _():
            fcb_ref[...] = fc_ref[0].astype(jnp.bfloat16)
            pjb_ref[...] = pj_ref[0].astype(jnp.bfloat16)

        h = jnp.dot(xs_ref[...].astype(jnp.bfloat16), fcb_ref[...].T,
                    preferred_element_type=jnp.float32)
        h = jnp.square(jnp.maximum(h, 0.0))
        y = jnp.dot(h.astype(jnp.bfloat16), pjb_ref[...].T,
                    preferred_element_type=jnp.float32)
        ys_ref[...] = y


def _run_mlp(te, tt, nt, xs, fc_w, proj_w):
    def tile_map(g, te_r, tt_r, nt_r):
        return (te_r[g] * NTILES + tt_r[g], 0)

    grid_spec = pltpu.PrefetchScalarGridSpec(
        num_scalar_prefetch=3,
        grid=(GMAX,),
        in_specs=[
            pl.BlockSpec((TILE, DIM), tile_map),
            pl.BlockSpec((1, HID, DIM), lambda g, te_r, tt_r, nt_r: (te_r[g], 0, 0)),
            pl.BlockSpec((1, DIM, HID), lambda g, te_r, tt_r, nt_r: (te_r[g], 0, 0)),
        ],
        out_specs=pl.BlockSpec((TILE, DIM), tile_map),
        scratch_shapes=[
            pltpu.VMEM((HID, DIM), jnp.bfloat16),
            pltpu.VMEM((DIM, HID), jnp.bfloat16),
        ],
    )
    return pl.pallas_call(
        _mlp_kernel,
        grid_spec=grid_spec,
        out_shape=jax.ShapeDtypeStruct((E * CAP, DIM), jnp.float32),
        compiler_params=pltpu.CompilerParams(
            dimension_semantics=("arbitrary",),
        ),
    )(te, tt, nt, xs, fc_w, proj_w)


_SC_CHUNK = 16
_NCHUNK = PER_W // _SC_CHUNK


def _run_combine(ys, s1, s2, w1b, w2b):
    mesh = plsc.VectorSubcoreMesh(core_axis_name="c", subcore_axis_name="s")

    @functools.partial(
        pl.kernel,
        mesh=mesh,
        out_type=jax.ShapeDtypeStruct((T, DIM), jnp.float32),
        scratch_types=[
            pltpu.VMEM((PER_W,), jnp.int32),
            pltpu.VMEM((PER_W,), jnp.int32),
            pltpu.VMEM((PER_W, 16), jnp.float32),
            pltpu.VMEM((PER_W, 16), jnp.float32),
            pltpu.VMEM((2, _SC_CHUNK, DIM), jnp.float32),
            pltpu.VMEM((2, _SC_CHUNK, DIM), jnp.float32),
            pltpu.SemaphoreType.DMA,
            pltpu.SemaphoreType.DMA,
        ],
    )
    def combine(ys_hbm, s1_hbm, s2_hbm, w1_hbm, w2_hbm, out_hbm,
                i1_v, i2_v, wa_v, wb_v, ra_v, rb_v, sem0, sem1):
        wid = lax.axis_index("s") * 2 + lax.axis_index("c")
        base = wid * PER_W
        pltpu.sync_copy(s1_hbm.at[pl.ds(base, PER_W)], i1_v)
        pltpu.sync_copy(s2_hbm.at[pl.ds(base, PER_W)], i2_v)
        pltpu.sync_copy(w1_hbm.at[pl.ds(base, PER_W)], wa_v)
        pltpu.sync_copy(w2_hbm.at[pl.ds(base, PER_W)], wb_v)
        sems = (sem0, sem1)

        def issue(c):
            slot = c % 2
            ca = pltpu.async_copy(
                ys_hbm.at[i1_v.at[pl.ds(c * _SC_CHUNK, _SC_CHUNK)]],
                ra_v.at[slot], sems[slot])
            cb = pltpu.async_copy(
                ys_hbm.at[i2_v.at[pl.ds(c * _SC_CHUNK, _SC_CHUNK)]],
                rb_v.at[slot], sems[slot])
            return ca, cb

        pend = {0: issue(0), 1: issue(1)}
        for c in range(_NCHUNK):
            slot = c % 2
            ca, cb = pend.pop(c)
            ca.wait()
            cb.wait()

            @pl.loop(0, _SC_CHUNK)
            def _(r):
                wa = wa_v.at[c * _SC_CHUNK + r][...]
                wb = wb_v.at[c * _SC_CHUNK + r][...]

                @pl.loop(0, DIM // 16)
                def _(i):
                    sl = (slot, r, pl.ds(i * 16, 16))
                    ra_v.at[*sl][...] = (ra_v.at[*sl][...] * wa
                                         + rb_v.at[*sl][...] * wb)

            pltpu.sync_copy(ra_v.at[slot],
                            out_hbm.at[pl.ds(base + c * _SC_CHUNK, _SC_CHUNK)])
            if c + 2 < _NCHUNK:
                pend[c + 2] = issue(c + 2)

    return combine(ys, s1, s2, w1b, w2b)


def kernel(x, router_w, fc_w, proj_w):
    bsz, seq_len, dim = x.shape
    x_flat = x.reshape(-1, dim)
    s1, s2, w1b, w2b, te, tt, nt = _run_router(x_flat, router_w)
    s1f = s1.reshape(T)
    s2f = s2.reshape(T)
    xs = _run_dispatch(x_flat, s1f, s2f)
    ys = _run_mlp(te.reshape(GMAX), tt.reshape(GMAX), nt.reshape(1), xs,
                  fc_w, proj_w)
    out = _run_combine(ys, s1f, s2f, w1b.reshape(T, 16), w2b.reshape(T, 16))
    return out.reshape(bsz, seq_len, dim), jnp.float32(0.0)


# speedup vs baseline: 2.8808x; 1.0369x over previous
import functools

import jax
import jax.numpy as jnp
from jax import lax
from jax.experimental import pallas as pl
from jax.experimental.pallas import tpu as pltpu
from jax.experimental.pallas import tpu_sc as plsc

DIM = 1024
HID = 2048
E = 8
T = 2048
BT = 256
TILE = 256
CAP = T
NTILES = CAP // TILE
GMAX = 23
NW = 32
PER_W = T // NW


def _router_kernel(x_ref, rw_ref, s1_ref, s2_ref, w1_ref, w2_ref,
                   te_ref, tt_ref, nt_ref, carry_ref):
    b = pl.program_id(0)

    @pl.when(b == 0)
    def _():
        carry_ref[...] = jnp.zeros_like(carry_ref)

    xb = x_ref[...]
    logits = jnp.dot(xb, rw_ref[...].T, preferred_element_type=jnp.float32)
    eidx = lax.broadcasted_iota(jnp.int32, logits.shape, 1)
    m1 = jnp.max(logits, axis=1, keepdims=True)
    i1 = jnp.min(jnp.where(logits == m1, eidx, E), axis=1, keepdims=True)
    masked = jnp.where(eidx == i1, -jnp.inf, logits)
    m2 = jnp.max(masked, axis=1, keepdims=True)
    i2 = jnp.min(jnp.where(masked == m2, eidx, E), axis=1, keepdims=True)
    denom = jnp.sum(jnp.exp(logits - m1), axis=1, keepdims=True)
    p1 = 1.0 / denom
    p2 = jnp.exp(m2 - m1) / denom
    s = p1 + p2 + 1e-8
    w1_ref[0] = jnp.broadcast_to(p1 / s, (BT, 16))
    w2_ref[0] = jnp.broadcast_to(p2 / s, (BT, 16))

    sel1 = (eidx == i1).astype(jnp.float32)
    sel2 = (eidx == i2).astype(jnp.float32)
    sel = sel1 + sel2
    ri = lax.broadcasted_iota(jnp.int32, (BT, BT), 0)
    ci = lax.broadcasted_iota(jnp.int32, (BT, BT), 1)
    ltri = (ri > ci).astype(jnp.bfloat16)
    pos = jnp.dot(ltri, sel.astype(jnp.bfloat16),
                  preferred_element_type=jnp.float32)
    pos = pos + carry_ref[...]
    base = (eidx * CAP).astype(jnp.float32)
    slotf = base + pos
    s1_ref[0] = jnp.sum(sel1 * slotf, axis=1, keepdims=True).astype(jnp.int32)
    s2_ref[0] = jnp.sum(sel2 * slotf, axis=1, keepdims=True).astype(jnp.int32)
    carry_ref[...] += jnp.sum(sel, axis=0, keepdims=True)

    @pl.when(b == pl.num_programs(0) - 1)
    def _():
        counts = carry_ref[...]
        nt = jnp.floor((counts + (TILE - 1)) * (1.0 / TILE))
        ut = (lax.broadcasted_iota(jnp.int32, (E, E), 0)
              <= lax.broadcasted_iota(jnp.int32, (E, E), 1))
        cume = jnp.dot(nt.astype(jnp.bfloat16), ut.astype(jnp.bfloat16),
                       preferred_element_type=jnp.float32)
        cums = cume - nt
        g_row = lax.broadcasted_iota(jnp.int32, (1, GMAX), 1).astype(jnp.float32)
        te = jnp.zeros((1, GMAX), jnp.float32)
        for e in range(E):
            te = te + (g_row >= cume[0, e]).astype(jnp.float32)
        te = jnp.minimum(te, float(E - 1))
        tt = g_row
        for e in range(E):
            tt = tt - jnp.where(te == e, cums[0, e], 0.0)
        tt = jnp.clip(tt, 0.0, float(NTILES - 1))
        ntot = cume[0, E - 1]
        last = jnp.maximum(ntot - 1.0, 0.0)
        te_last = jnp.sum(jnp.where(g_row == last, te, 0.0), axis=1,
                          keepdims=True)
        tt_last = jnp.sum(jnp.where(g_row == last, tt, 0.0), axis=1,
                          keepdims=True)
        active = g_row < ntot
        te_ref[...] = jnp.where(active, te, te_last).astype(jnp.int32)
        tt_ref[...] = jnp.where(active, tt, tt_last).astype(jnp.int32)
        nt_ref[...] = jnp.full((1, 1), ntot, jnp.float32).astype(jnp.int32)


def _run_router(x_flat, router_w):
    nb = T // BT
    return pl.pallas_call(
        _router_kernel,
        grid=(nb,),
        in_specs=[
            pl.BlockSpec((BT, DIM), lambda b: (b, 0)),
            pl.BlockSpec((E, DIM), lambda b: (0, 0)),
        ],
        out_specs=[
            pl.BlockSpec((1, BT, 1), lambda b: (b, 0, 0)),
            pl.BlockSpec((1, BT, 1), lambda b: (b, 0, 0)),
            pl.BlockSpec((1, BT, 16), lambda b: (b, 0, 0)),
            pl.BlockSpec((1, BT, 16), lambda b: (b, 0, 0)),
            pl.BlockSpec((1, GMAX), lambda b: (0, 0)),
            pl.BlockSpec((1, GMAX), lambda b: (0, 0)),
            pl.BlockSpec((1, 1), lambda b: (0, 0)),
        ],
        out_shape=[
            jax.ShapeDtypeStruct((nb, BT, 1), jnp.int32),
            jax.ShapeDtypeStruct((nb, BT, 1), jnp.int32),
            jax.ShapeDtypeStruct((nb, BT, 16), jnp.float32),
            jax.ShapeDtypeStruct((nb, BT, 16), jnp.float32),
            jax.ShapeDtypeStruct((1, GMAX), jnp.int32),
            jax.ShapeDtypeStruct((1, GMAX), jnp.int32),
            jax.ShapeDtypeStruct((1, 1), jnp.int32),
        ],
        scratch_shapes=[pltpu.VMEM((1, E), jnp.float32)],
        compiler_params=pltpu.CompilerParams(
            dimension_semantics=("arbitrary",),
        ),
    )(x_flat, router_w)


def _run_dispatch(x_flat, s1, s2):
    mesh = plsc.VectorSubcoreMesh(core_axis_name="c", subcore_axis_name="s")

    @functools.partial(
        pl.kernel,
        mesh=mesh,
        out_type=jax.ShapeDtypeStruct((E * CAP, DIM), jnp.float32),
        scratch_types=[
            pltpu.VMEM((PER_W,), jnp.int32),
            pltpu.VMEM((PER_W,), jnp.int32),
            pltpu.VMEM((PER_W, DIM), jnp.float32),
            pltpu.SemaphoreType.DMA,
        ],
    )
    def dispatch(x_hbm, s1_hbm, s2_hbm, xs_hbm, i1_v, i2_v, xr_v, sem):
        wid = lax.axis_index("s") * 2 + lax.axis_index("c")
        base = wid * PER_W
        pltpu.sync_copy(s1_hbm.at[pl.ds(base, PER_W)], i1_v)
        pltpu.sync_copy(s2_hbm.at[pl.ds(base, PER_W)], i2_v)
        pltpu.sync_copy(x_hbm.at[pl.ds(base, PER_W)], xr_v)
        c1 = pltpu.async_copy(xr_v, xs_hbm.at[i1_v], sem)
        c2 = pltpu.async_copy(xr_v, xs_hbm.at[i2_v], sem)
        c1.wait()
        c2.wait()

    return dispatch(x_flat, s1, s2)


def _mlp_kernel(te_ref, tt_ref, nt_ref, xs_ref, fc_ref, pj_ref, ys_ref):
    g = pl.program_id(0)

    @pl.when(g < nt_ref[0])
    def _():
        h = jnp.dot(xs_ref[...], fc_ref[0].T,
                    preferred_element_type=jnp.float32)
        h = jnp.square(jnp.maximum(h, 0.0))
        y = jnp.dot(h, pj_ref[0].T, preferred_element_type=jnp.float32)
        ys_ref[...] = y


def _run_mlp(te, tt, nt, xs, fc_w, proj_w):
    def tile_map(g, te_r, tt_r, nt_r):
        return (te_r[g] * NTILES + tt_r[g], 0)

    grid_spec = pltpu.PrefetchScalarGridSpec(
        num_scalar_prefetch=3,
        grid=(GMAX,),
        in_specs=[
            pl.BlockSpec((TILE, DIM), tile_map),
            pl.BlockSpec((1, HID, DIM), lambda g, te_r, tt_r, nt_r: (te_r[g], 0, 0)),
            pl.BlockSpec((1, DIM, HID), lambda g, te_r, tt_r, nt_r: (te_r[g], 0, 0)),
        ],
        out_specs=pl.BlockSpec((TILE, DIM), tile_map),
    )
    return pl.pallas_call(
        _mlp_kernel,
        grid_spec=grid_spec,
        out_shape=jax.ShapeDtypeStruct((E * CAP, DIM), jnp.float32),
        compiler_params=pltpu.CompilerParams(
            dimension_semantics=("arbitrary",),
        ),
    )(te, tt, nt, xs, fc_w, proj_w)


_SC_CHUNK = 16
_NCHUNK = PER_W // _SC_CHUNK


def _run_combine(ys, s1, s2, w1b, w2b):
    mesh = plsc.VectorSubcoreMesh(core_axis_name="c", subcore_axis_name="s")

    @functools.partial(
        pl.kernel,
        mesh=mesh,
        out_type=jax.ShapeDtypeStruct((T, DIM), jnp.float32),
        scratch_types=[
            pltpu.VMEM((PER_W,), jnp.int32),
            pltpu.VMEM((PER_W,), jnp.int32),
            pltpu.VMEM((PER_W, 16), jnp.float32),
            pltpu.VMEM((PER_W, 16), jnp.float32),
            pltpu.VMEM((2, _SC_CHUNK, DIM), jnp.float32),
            pltpu.VMEM((2, _SC_CHUNK, DIM), jnp.float32),
            pltpu.SemaphoreType.DMA,
            pltpu.SemaphoreType.DMA,
        ],
    )
    def combine(ys_hbm, s1_hbm, s2_hbm, w1_hbm, w2_hbm, out_hbm,
                i1_v, i2_v, wa_v, wb_v, ra_v, rb_v, sem0, sem1):
        wid = lax.axis_index("s") * 2 + lax.axis_index("c")
        base = wid * PER_W
        pltpu.sync_copy(s1_hbm.at[pl.ds(base, PER_W)], i1_v)
        pltpu.sync_copy(s2_hbm.at[pl.ds(base, PER_W)], i2_v)
        pltpu.sync_copy(w1_hbm.at[pl.ds(base, PER_W)], wa_v)
        pltpu.sync_copy(w2_hbm.at[pl.ds(base, PER_W)], wb_v)
        sems = (sem0, sem1)

        def issue(c):
            slot = c % 2
            ca = pltpu.async_copy(
                ys_hbm.at[i1_v.at[pl.ds(c * _SC_CHUNK, _SC_CHUNK)]],
                ra_v.at[slot], sems[slot])
            cb = pltpu.async_copy(
                ys_hbm.at[i2_v.at[pl.ds(c * _SC_CHUNK, _SC_CHUNK)]],
                rb_v.at[slot], sems[slot])
            return ca, cb

        pend = {0: issue(0), 1: issue(1)}
        for c in range(_NCHUNK):
            slot = c % 2
            ca, cb = pend.pop(c)
            ca.wait()
            cb.wait()

            @pl.loop(0, _SC_CHUNK)
            def _(r):
                wa = wa_v.at[c * _SC_CHUNK + r][...]
                wb = wb_v.at[c * _SC_CHUNK + r][...]

                @pl.loop(0, DIM // 16)
                def _(i):
                    sl = (slot, r, pl.ds(i * 16, 16))
                    ra_v.at[*sl][...] = (ra_v.at[*sl][...] * wa
                                         + rb_v.at[*sl][...] * wb)

            pltpu.sync_copy(ra_v.at[slot],
                            out_hbm.at[pl.ds(base + c * _SC_CHUNK, _SC_CHUNK)])
            if c + 2 < _NCHUNK:
                pend[c + 2] = issue(c + 2)

    return combine(ys, s1, s2, w1b, w2b)


def kernel(x, router_w, fc_w, proj_w):
    bsz, seq_len, dim = x.shape
    x_flat = x.reshape(-1, dim)
    s1, s2, w1b, w2b, te, tt, nt = _run_router(x_flat, router_w)
    s1f = s1.reshape(T)
    s2f = s2.reshape(T)
    xs = _run_dispatch(x_flat, s1f, s2f)
    ys = _run_mlp(te.reshape(GMAX), tt.reshape(GMAX), nt.reshape(1), xs,
                  fc_w, proj_w)
    out = _run_combine(ys, s1f, s2f, w1b.reshape(T, 16), w2b.reshape(T, 16))
    return out.reshape(bsz, seq_len, dim), jnp.float32(0.0)


# speedup vs baseline: 3.1626x; 1.0978x over previous
import functools

import jax
import jax.numpy as jnp
from jax import lax
from jax.experimental import pallas as pl
from jax.experimental.pallas import tpu as pltpu
from jax.experimental.pallas import tpu_sc as plsc

DIM = 1024
HID = 2048
E = 8
T = 2048
BT = 256
TILE = 512
CAP = T
NTILES = CAP // TILE
GMAX = 15
NW = 32
PER_W = T // NW


def _router_kernel(x_ref, rw_ref, s1_ref, s2_ref, w1_ref, w2_ref,
                   te_ref, tt_ref, nt_ref, carry_ref):
    b = pl.program_id(0)

    @pl.when(b == 0)
    def _():
        carry_ref[...] = jnp.zeros_like(carry_ref)

    xb = x_ref[...]
    logits = jnp.dot(xb, rw_ref[...].T, preferred_element_type=jnp.float32)
    eidx = lax.broadcasted_iota(jnp.int32, logits.shape, 1)
    m1 = jnp.max(logits, axis=1, keepdims=True)
    i1 = jnp.min(jnp.where(logits == m1, eidx, E), axis=1, keepdims=True)
    masked = jnp.where(eidx == i1, -jnp.inf, logits)
    m2 = jnp.max(masked, axis=1, keepdims=True)
    i2 = jnp.min(jnp.where(masked == m2, eidx, E), axis=1, keepdims=True)
    denom = jnp.sum(jnp.exp(logits - m1), axis=1, keepdims=True)
    p1 = 1.0 / denom
    p2 = jnp.exp(m2 - m1) / denom
    s = p1 + p2 + 1e-8
    w1_ref[0] = jnp.broadcast_to(p1 / s, (BT, 16))
    w2_ref[0] = jnp.broadcast_to(p2 / s, (BT, 16))

    sel1 = (eidx == i1).astype(jnp.float32)
    sel2 = (eidx == i2).astype(jnp.float32)
    sel = sel1 + sel2
    ri = lax.broadcasted_iota(jnp.int32, (BT, BT), 0)
    ci = lax.broadcasted_iota(jnp.int32, (BT, BT), 1)
    ltri = (ri > ci).astype(jnp.bfloat16)
    pos = jnp.dot(ltri, sel.astype(jnp.bfloat16),
                  preferred_element_type=jnp.float32)
    pos = pos + carry_ref[...]
    base = (eidx * CAP).astype(jnp.float32)
    slotf = base + pos
    s1_ref[0] = jnp.sum(sel1 * slotf, axis=1, keepdims=True).astype(jnp.int32)
    s2_ref[0] = jnp.sum(sel2 * slotf, axis=1, keepdims=True).astype(jnp.int32)
    carry_ref[...] += jnp.sum(sel, axis=0, keepdims=True)

    @pl.when(b == pl.num_programs(0) - 1)
    def _():
        counts = carry_ref[...]
        nt = jnp.floor((counts + (TILE - 1)) * (1.0 / TILE))
        ut = (lax.broadcasted_iota(jnp.int32, (E, E), 0)
              <= lax.broadcasted_iota(jnp.int32, (E, E), 1))
        cume = jnp.dot(nt.astype(jnp.bfloat16), ut.astype(jnp.bfloat16),
                       preferred_element_type=jnp.float32)
        cums = cume - nt
        g_row = lax.broadcasted_iota(jnp.int32, (1, GMAX), 1).astype(jnp.float32)
        te = jnp.zeros((1, GMAX), jnp.float32)
        for e in range(E):
            te = te + (g_row >= cume[0, e]).astype(jnp.float32)
        te = jnp.minimum(te, float(E - 1))
        tt = g_row
        for e in range(E):
            tt = tt - jnp.where(te == e, cums[0, e], 0.0)
        tt = jnp.clip(tt, 0.0, float(NTILES - 1))
        ntot = cume[0, E - 1]
        last = jnp.maximum(ntot - 1.0, 0.0)
        te_last = jnp.sum(jnp.where(g_row == last, te, 0.0), axis=1,
                          keepdims=True)
        tt_last = jnp.sum(jnp.where(g_row == last, tt, 0.0), axis=1,
                          keepdims=True)
        active = g_row < ntot
        te_ref[...] = jnp.where(active, te, te_last).astype(jnp.int32)
        tt_ref[...] = jnp.where(active, tt, tt_last).astype(jnp.int32)
        nt_ref[...] = jnp.full((1, 1), ntot, jnp.float32).astype(jnp.int32)


def _run_router(x_flat, router_w):
    nb = T // BT
    return pl.pallas_call(
        _router_kernel,
        grid=(nb,),
        in_specs=[
            pl.BlockSpec((BT, DIM), lambda b: (b, 0)),
            pl.BlockSpec((E, DIM), lambda b: (0, 0)),
        ],
        out_specs=[
            pl.BlockSpec((1, BT, 1), lambda b: (b, 0, 0)),
            pl.BlockSpec((1, BT, 1), lambda b: (b, 0, 0)),
            pl.BlockSpec((1, BT, 16), lambda b: (b, 0, 0)),
            pl.BlockSpec((1, BT, 16), lambda b: (b, 0, 0)),
            pl.BlockSpec((1, GMAX), lambda b: (0, 0)),
            pl.BlockSpec((1, GMAX), lambda b: (0, 0)),
            pl.BlockSpec((1, 1), lambda b: (0, 0)),
        ],
        out_shape=[
            jax.ShapeDtypeStruct((nb, BT, 1), jnp.int32),
            jax.ShapeDtypeStruct((nb, BT, 1), jnp.int32),
            jax.ShapeDtypeStruct((nb, BT, 16), jnp.float32),
            jax.ShapeDtypeStruct((nb, BT, 16), jnp.float32),
            jax.ShapeDtypeStruct((1, GMAX), jnp.int32),
            jax.ShapeDtypeStruct((1, GMAX), jnp.int32),
            jax.ShapeDtypeStruct((1, 1), jnp.int32),
        ],
        scratch_shapes=[pltpu.VMEM((1, E), jnp.float32)],
        compiler_params=pltpu.CompilerParams(
            dimension_semantics=("arbitrary",),
        ),
    )(x_flat, router_w)


def _run_dispatch(x_flat, s1, s2):
    mesh = plsc.VectorSubcoreMesh(core_axis_name="c", subcore_axis_name="s")

    @functools.partial(
        pl.kernel,
        mesh=mesh,
        out_type=jax.ShapeDtypeStruct((E * CAP, DIM), jnp.float32),
        scratch_types=[
            pltpu.VMEM((PER_W,), jnp.int32),
            pltpu.VMEM((PER_W,), jnp.int32),
            pltpu.VMEM((PER_W, DIM), jnp.float32),
            pltpu.SemaphoreType.DMA,
        ],
    )
    def dispatch(x_hbm, s1_hbm, s2_hbm, xs_hbm, i1_v, i2_v, xr_v, sem):
        wid = lax.axis_index("s") * 2 + lax.axis_index("c")
        base = wid * PER_W
        pltpu.sync_copy(s1_hbm.at[pl.ds(base, PER_W)], i1_v)
        pltpu.sync_copy(s2_hbm.at[pl.ds(base, PER_W)], i2_v)
        pltpu.sync_copy(x_hbm.at[pl.ds(base, PER_W)], xr_v)
        c1 = pltpu.async_copy(xr_v, xs_hbm.at[i1_v], sem)
        c2 = pltpu.async_copy(xr_v, xs_hbm.at[i2_v], sem)
        c1.wait()
        c2.wait()

    return dispatch(x_flat, s1, s2)


def _mlp_kernel(te_ref, tt_ref, nt_ref, xs_ref, fc_ref, pj_ref, ys_ref):
    g = pl.program_id(0)

    @pl.when(g < nt_ref[0])
    def _():
        h = jnp.dot(xs_ref[...], fc_ref[0].T,
                    preferred_element_type=jnp.float32)
        h = jnp.square(jnp.maximum(h, 0.0))
        y = jnp.dot(h, pj_ref[0].T, preferred_element_type=jnp.float32)
        ys_ref[...] = y


def _run_mlp(te, tt, nt, xs, fc_w, proj_w):
    def tile_map(g, te_r, tt_r, nt_r):
        return (te_r[g] * NTILES + tt_r[g], 0)

    grid_spec = pltpu.PrefetchScalarGridSpec(
        num_scalar_prefetch=3,
        grid=(GMAX,),
        in_specs=[
            pl.BlockSpec((TILE, DIM), tile_map),
            pl.BlockSpec((1, HID, DIM), lambda g, te_r, tt_r, nt_r: (te_r[g], 0, 0)),
            pl.BlockSpec((1, DIM, HID), lambda g, te_r, tt_r, nt_r: (te_r[g], 0, 0)),
        ],
        out_specs=pl.BlockSpec((TILE, DIM), tile_map),
    )
    return pl.pallas_call(
        _mlp_kernel,
        grid_spec=grid_spec,
        out_shape=jax.ShapeDtypeStruct((E * CAP, DIM), jnp.float32),
        compiler_params=pltpu.CompilerParams(
            dimension_semantics=("arbitrary",),
        ),
    )(te, tt, nt, xs, fc_w, proj_w)


_SC_CHUNK = 16
_NCHUNK = PER_W // _SC_CHUNK


def _run_combine(ys, s1, s2, w1b, w2b):
    mesh = plsc.VectorSubcoreMesh(core_axis_name="c", subcore_axis_name="s")

    @functools.partial(
        pl.kernel,
        mesh=mesh,
        out_type=jax.ShapeDtypeStruct((T, DIM), jnp.float32),
        scratch_types=[
            pltpu.VMEM((PER_W,), jnp.int32),
            pltpu.VMEM((PER_W,), jnp.int32),
            pltpu.VMEM((PER_W, 16), jnp.float32),
            pltpu.VMEM((PER_W, 16), jnp.float32),
            pltpu.VMEM((2, _SC_CHUNK, DIM), jnp.float32),
            pltpu.VMEM((2, _SC_CHUNK, DIM), jnp.float32),
            pltpu.SemaphoreType.DMA,
            pltpu.SemaphoreType.DMA,
        ],
    )
    def combine(ys_hbm, s1_hbm, s2_hbm, w1_hbm, w2_hbm, out_hbm,
                i1_v, i2_v, wa_v, wb_v, ra_v, rb_v, sem0, sem1):
        wid = lax.axis_index("s") * 2 + lax.axis_index("c")
        base = wid * PER_W
        pltpu.sync_copy(s1_hbm.at[pl.ds(base, PER_W)], i1_v)
        pltpu.sync_copy(s2_hbm.at[pl.ds(base, PER_W)], i2_v)
        pltpu.sync_copy(w1_hbm.at[pl.ds(base, PER_W)], wa_v)
        pltpu.sync_copy(w2_hbm.at[pl.ds(base, PER_W)], wb_v)
        sems = (sem0, sem1)

        def issue(c):
            slot = c % 2
            ca = pltpu.async_copy(
                ys_hbm.at[i1_v.at[pl.ds(c * _SC_CHUNK, _SC_CHUNK)]],
                ra_v.at[slot], sems[slot])
            cb = pltpu.async_copy(
                ys_hbm.at[i2_v.at[pl.ds(c * _SC_CHUNK, _SC_CHUNK)]],
                rb_v.at[slot], sems[slot])
            return ca, cb

        pend = {0: issue(0), 1: issue(1)}
        for c in range(_NCHUNK):
            slot = c % 2
            ca, cb = pend.pop(c)
            ca.wait()
            cb.wait()

            @pl.loop(0, _SC_CHUNK)
            def _(r):
                wa = wa_v.at[c * _SC_CHUNK + r][...]
                wb = wb_v.at[c * _SC_CHUNK + r][...]

                @pl.loop(0, DIM // 16)
                def _(i):
                    sl = (slot, r, pl.ds(i * 16, 16))
                    ra_v.at[*sl][...] = (ra_v.at[*sl][...] * wa
                                         + rb_v.at[*sl][...] * wb)

            pltpu.sync_copy(ra_v.at[slot],
                            out_hbm.at[pl.ds(base + c * _SC_CHUNK, _SC_CHUNK)])
            if c + 2 < _NCHUNK:
                pend[c + 2] = issue(c + 2)

    return combine(ys, s1, s2, w1b, w2b)


def kernel(x, router_w, fc_w, proj_w):
    bsz, seq_len, dim = x.shape
    x_flat = x.reshape(-1, dim)
    s1, s2, w1b, w2b, te, tt, nt = _run_router(x_flat, router_w)
    s1f = s1.reshape(T)
    s2f = s2.reshape(T)
    xs = _run_dispatch(x_flat, s1f, s2f)
    ys = _run_mlp(te.reshape(GMAX), tt.reshape(GMAX), nt.reshape(1), xs,
                  fc_w, proj_w)
    out = _run_combine(ys, s1f, s2f, w1b.reshape(T, 16), w2b.reshape(T, 16))
    return out.reshape(bsz, seq_len, dim), jnp.float32(0.0)


# speedup vs baseline: 3.2289x; 1.0209x over previous
import functools

import jax
import jax.numpy as jnp
from jax import lax
from jax.experimental import pallas as pl
from jax.experimental.pallas import tpu as pltpu
from jax.experimental.pallas import tpu_sc as plsc

DIM = 1024
HID = 2048
E = 8
T = 2048
BT = 512
TILE = 512
CAP = T
NTILES = CAP // TILE
GMAX = 15
NW = 32
PER_W = T // NW


def _router_kernel(x_ref, rw_ref, s1_ref, s2_ref, w1_ref, w2_ref,
                   te_ref, tt_ref, nt_ref, carry_ref):
    b = pl.program_id(0)

    @pl.when(b == 0)
    def _():
        carry_ref[...] = jnp.zeros_like(carry_ref)

    xb = x_ref[...]
    logits = jnp.dot(xb, rw_ref[...].T, preferred_element_type=jnp.float32)
    eidx = lax.broadcasted_iota(jnp.int32, logits.shape, 1)
    m1 = jnp.max(logits, axis=1, keepdims=True)
    i1 = jnp.min(jnp.where(logits == m1, eidx, E), axis=1, keepdims=True)
    masked = jnp.where(eidx == i1, -jnp.inf, logits)
    m2 = jnp.max(masked, axis=1, keepdims=True)
    i2 = jnp.min(jnp.where(masked == m2, eidx, E), axis=1, keepdims=True)
    denom = jnp.sum(jnp.exp(logits - m1), axis=1, keepdims=True)
    p1 = 1.0 / denom
    p2 = jnp.exp(m2 - m1) / denom
    s = p1 + p2 + 1e-8
    w1_ref[0] = jnp.broadcast_to(p1 / s, (BT, 16))
    w2_ref[0] = jnp.broadcast_to(p2 / s, (BT, 16))

    sel1 = (eidx == i1).astype(jnp.float32)
    sel2 = (eidx == i2).astype(jnp.float32)
    sel = sel1 + sel2
    ri = lax.broadcasted_iota(jnp.int32, (BT, BT), 0)
    ci = lax.broadcasted_iota(jnp.int32, (BT, BT), 1)
    ltri = (ri > ci).astype(jnp.bfloat16)
    pos = jnp.dot(ltri, sel.astype(jnp.bfloat16),
                  preferred_element_type=jnp.float32)
    pos = pos + carry_ref[...]
    base = (eidx * CAP).astype(jnp.float32)
    slotf = base + pos
    s1_ref[0] = jnp.sum(sel1 * slotf, axis=1, keepdims=True).astype(jnp.int32)
    s2_ref[0] = jnp.sum(sel2 * slotf, axis=1, keepdims=True).astype(jnp.int32)
    carry_ref[...] += jnp.sum(sel, axis=0, keepdims=True)

    @pl.when(b == pl.num_programs(0) - 1)
    def _():
        counts = carry_ref[...]
        nt = jnp.floor((counts + (TILE - 1)) * (1.0 / TILE))
        ut = (lax.broadcasted_iota(jnp.int32, (E, E), 0)
              <= lax.broadcasted_iota(jnp.int32, (E, E), 1))
        cume = jnp.dot(nt.astype(jnp.bfloat16), ut.astype(jnp.bfloat16),
                       preferred_element_type=jnp.float32)
        cums = cume - nt
        g_row = lax.broadcasted_iota(jnp.int32, (1, GMAX), 1).astype(jnp.float32)
        te = jnp.zeros((1, GMAX), jnp.float32)
        for e in range(E):
            te = te + (g_row >= cume[0, e]).astype(jnp.float32)
        te = jnp.minimum(te, float(E - 1))
        tt = g_row
        for e in range(E):
            tt = tt - jnp.where(te == e, cums[0, e], 0.0)
        tt = jnp.clip(tt, 0.0, float(NTILES - 1))
        ntot = cume[0, E - 1]
        last = jnp.maximum(ntot - 1.0, 0.0)
        te_last = jnp.sum(jnp.where(g_row == last, te, 0.0), axis=1,
                          keepdims=True)
        tt_last = jnp.sum(jnp.where(g_row == last, tt, 0.0), axis=1,
                          keepdims=True)
        active = g_row < ntot
        te_ref[...] = jnp.where(active, te, te_last).astype(jnp.int32)
        tt_ref[...] = jnp.where(active, tt, tt_last).astype(jnp.int32)
        nt_ref[...] = jnp.full((1, 1), ntot, jnp.float32).astype(jnp.int32)


def _run_router(x_flat, router_w):
    nb = T // BT
    return pl.pallas_call(
        _router_kernel,
        grid=(nb,),
        in_specs=[
            pl.BlockSpec((BT, DIM), lambda b: (b, 0)),
            pl.BlockSpec((E, DIM), lambda b: (0, 0)),
        ],
        out_specs=[
            pl.BlockSpec((1, BT, 1), lambda b: (b, 0, 0)),
            pl.BlockSpec((1, BT, 1), lambda b: (b, 0, 0)),
            pl.BlockSpec((1, BT, 16), lambda b: (b, 0, 0)),
            pl.BlockSpec((1, BT, 16), lambda b: (b, 0, 0)),
            pl.BlockSpec((1, GMAX), lambda b: (0, 0)),
            pl.BlockSpec((1, GMAX), lambda b: (0, 0)),
            pl.BlockSpec((1, 1), lambda b: (0, 0)),
        ],
        out_shape=[
            jax.ShapeDtypeStruct((nb, BT, 1), jnp.int32),
            jax.ShapeDtypeStruct((nb, BT, 1), jnp.int32),
            jax.ShapeDtypeStruct((nb, BT, 16), jnp.float32),
            jax.ShapeDtypeStruct((nb, BT, 16), jnp.float32),
            jax.ShapeDtypeStruct((1, GMAX), jnp.int32),
            jax.ShapeDtypeStruct((1, GMAX), jnp.int32),
            jax.ShapeDtypeStruct((1, 1), jnp.int32),
        ],
        scratch_shapes=[pltpu.VMEM((1, E), jnp.float32)],
        compiler_params=pltpu.CompilerParams(
            dimension_semantics=("arbitrary",),
        ),
    )(x_flat, router_w)


def _run_dispatch(x_flat, s1, s2):
    mesh = plsc.VectorSubcoreMesh(core_axis_name="c", subcore_axis_name="s")

    @functools.partial(
        pl.kernel,
        mesh=mesh,
        out_type=jax.ShapeDtypeStruct((E * CAP, DIM), jnp.float32),
        scratch_types=[
            pltpu.VMEM((PER_W,), jnp.int32),
            pltpu.VMEM((PER_W,), jnp.int32),
            pltpu.VMEM((PER_W, DIM), jnp.float32),
            pltpu.SemaphoreType.DMA,
        ],
    )
    def dispatch(x_hbm, s1_hbm, s2_hbm, xs_hbm, i1_v, i2_v, xr_v, sem):
        wid = lax.axis_index("s") * 2 + lax.axis_index("c")
        base = wid * PER_W
        pltpu.sync_copy(s1_hbm.at[pl.ds(base, PER_W)], i1_v)
        pltpu.sync_copy(s2_hbm.at[pl.ds(base, PER_W)], i2_v)
        pltpu.sync_copy(x_hbm.at[pl.ds(base, PER_W)], xr_v)
        c1 = pltpu.async_copy(xr_v, xs_hbm.at[i1_v], sem)
        c2 = pltpu.async_copy(xr_v, xs_hbm.at[i2_v], sem)
        c1.wait()
        c2.wait()

    return dispatch(x_flat, s1, s2)


def _mlp_kernel(te_ref, tt_ref, nt_ref, xs_ref, fc_ref, pj_ref, ys_ref):
    g = pl.program_id(0)

    @pl.when(g < nt_ref[0])
    def _():
        h = jnp.dot(xs_ref[...], fc_ref[0].T,
                    preferred_element_type=jnp.float32)
        h = jnp.square(jnp.maximum(h, 0.0))
        y = jnp.dot(h, pj_ref[0].T, preferred_element_type=jnp.float32)
        ys_ref[...] = y


def _run_mlp(te, tt, nt, xs, fc_w, proj_w):
    def tile_map(g, te_r, tt_r, nt_r):
        return (te_r[g] * NTILES + tt_r[g], 0)

    grid_spec = pltpu.PrefetchScalarGridSpec(
        num_scalar_prefetch=3,
        grid=(GMAX,),
        in_specs=[
            pl.BlockSpec((TILE, DIM), tile_map),
            pl.BlockSpec((1, HID, DIM), lambda g, te_r, tt_r, nt_r: (te_r[g], 0, 0)),
            pl.BlockSpec((1, DIM, HID), lambda g, te_r, tt_r, nt_r: (te_r[g], 0, 0)),
        ],
        out_specs=pl.BlockSpec((TILE, DIM), tile_map),
    )
    return pl.pallas_call(
        _mlp_kernel,
        grid_spec=grid_spec,
        out_shape=jax.ShapeDtypeStruct((E * CAP, DIM), jnp.float32),
        compiler_params=pltpu.CompilerParams(
            dimension_semantics=("arbitrary",),
        ),
    )(te, tt, nt, xs, fc_w, proj_w)


_SC_CHUNK = 16
_NCHUNK = PER_W // _SC_CHUNK


def _run_combine(ys, s1, s2, w1b, w2b):
    mesh = plsc.VectorSubcoreMesh(core_axis_name="c", subcore_axis_name="s")

    @functools.partial(
        pl.kernel,
        mesh=mesh,
        out_type=jax.ShapeDtypeStruct((T, DIM), jnp.float32),
        scratch_types=[
            pltpu.VMEM((PER_W,), jnp.int32),
            pltpu.VMEM((PER_W,), jnp.int32),
            pltpu.VMEM((PER_W, 16), jnp.float32),
            pltpu.VMEM((PER_W, 16), jnp.float32),
            pltpu.VMEM((2, _SC_CHUNK, DIM), jnp.float32),
            pltpu.VMEM((2, _SC_CHUNK, DIM), jnp.float32),
            pltpu.SemaphoreType.DMA,
            pltpu.SemaphoreType.DMA,
        ],
    )
    def combine(ys_hbm, s1_hbm, s2_hbm, w1_hbm, w2_hbm, out_hbm,
                i1_v, i2_v, wa_v, wb_v, ra_v, rb_v, sem0, sem1):
        wid = lax.axis_index("s") * 2 + lax.axis_index("c")
        base = wid * PER_W
        pltpu.sync_copy(s1_hbm.at[pl.ds(base, PER_W)], i1_v)
        pltpu.sync_copy(s2_hbm.at[pl.ds(base, PER_W)], i2_v)
        pltpu.sync_copy(w1_hbm.at[pl.ds(base, PER_W)], wa_v)
        pltpu.sync_copy(w2_hbm.at[pl.ds(base, PER_W)], wb_v)
        sems = (sem0, sem1)

        def issue(c):
            slot = c % 2
            ca = pltpu.async_copy(
                ys_hbm.at[i1_v.at[pl.ds(c * _SC_CHUNK, _SC_CHUNK)]],
                ra_v.at[slot], sems[slot])
            cb = pltpu.async_copy(
                ys_hbm.at[i2_v.at[pl.ds(c * _SC_CHUNK, _SC_CHUNK)]],
                rb_v.at[slot], sems[slot])
            return ca, cb

        pend = {0: issue(0), 1: issue(1)}
        for c in range(_NCHUNK):
            slot = c % 2
            ca, cb = pend.pop(c)
            ca.wait()
            cb.wait()

            @pl.loop(0, _SC_CHUNK)
            def _(r):
                wa = wa_v.at[c * _SC_CHUNK + r][...]
                wb = wb_v.at[c * _SC_CHUNK + r][...]

                @pl.loop(0, DIM // 16)
                def _(i):
                    sl = (slot, r, pl.ds(i * 16, 16))
                    ra_v.at[*sl][...] = (ra_v.at[*sl][...] * wa
                                         + rb_v.at[*sl][...] * wb)

            pltpu.sync_copy(ra_v.at[slot],
                            out_hbm.at[pl.ds(base + c * _SC_CHUNK, _SC_CHUNK)])
            if c + 2 < _NCHUNK:
                pend[c + 2] = issue(c + 2)

    return combine(ys, s1, s2, w1b, w2b)


def kernel(x, router_w, fc_w, proj_w):
    bsz, seq_len, dim = x.shape
    x_flat = x.reshape(-1, dim)
    s1, s2, w1b, w2b, te, tt, nt = _run_router(x_flat, router_w)
    s1f = s1.reshape(T)
    s2f = s2.reshape(T)
    xs = _run_dispatch(x_flat, s1f, s2f)
    ys = _run_mlp(te.reshape(GMAX), tt.reshape(GMAX), nt.reshape(1), xs,
                  fc_w, proj_w)
    out = _run_combine(ys, s1f, s2f, w1b.reshape(T, 16), w2b.reshape(T, 16))
    return out.reshape(bsz, seq_len, dim), jnp.float32(0.0)
